# Initial kernel scaffold; baseline (speedup 1.0000x reference)
import jax
import jax.numpy as jnp
from jax.experimental import pallas as pl


def kernel(edge_attr, W0, W1, W2):
    raise NotImplementedError("write your pallas kernel here")


# speedup vs baseline: 1.0830x; 1.0830x over previous
import functools

import jax
import jax.numpy as jnp
from jax import lax
from jax.experimental import pallas as pl
from jax.experimental.pallas import tpu as pltpu
from jax.experimental.pallas import tpu_sc as plsc

D_EMB = 128
NUM_WORKERS = 32
CHUNK = 80


def _bond_encode_sc(codes0, codes1, codes2, table, E, n1, n2):
    per_w = E // NUM_WORKERS
    n_chunks = per_w // CHUNK
    mesh = plsc.VectorSubcoreMesh(core_axis_name="c", subcore_axis_name="s")

    @functools.partial(
        pl.kernel,
        mesh=mesh,
        out_type=jax.ShapeDtypeStruct((E, D_EMB), jnp.float32),
        scratch_types=[
            pltpu.VMEM((CHUNK,), jnp.int32),
            pltpu.VMEM((CHUNK,), jnp.int32),
            pltpu.VMEM((CHUNK,), jnp.int32),
            pltpu.VMEM((CHUNK,), jnp.int32),
            pltpu.VMEM((CHUNK, D_EMB), jnp.float32),
            pltpu.SemaphoreType.DMA,
        ],
    )
    def k(c0_h, c1_h, c2_h, s_h, out_h, a0_v, a1_v, a2_v, idx_v, rows_v, sem):
        wid = lax.axis_index("s") * 2 + lax.axis_index("c")
        base_w = wid * per_w

        def chunk_body(ci, carry):
            base = base_w + ci * CHUNK
            pltpu.sync_copy(c0_h.at[pl.ds(base, CHUNK)], a0_v)
            pltpu.sync_copy(c1_h.at[pl.ds(base, CHUNK)], a1_v)
            pltpu.sync_copy(c2_h.at[pl.ds(base, CHUNK)], a2_v)
            for i in range(CHUNK // 16):
                s = pl.ds(i * 16, 16)
                idx_v[s] = a0_v[s] * (n1 * n2) + a1_v[s] * n2 + a2_v[s]
            pltpu.async_copy(s_h.at[idx_v], rows_v, sem).wait()
            pltpu.sync_copy(rows_v, out_h.at[pl.ds(base, CHUNK)])
            return carry

        lax.fori_loop(0, n_chunks, chunk_body, 0)

    return k(codes0, codes1, codes2, table)


def kernel(edge_attr, W0, W1, W2):
    E = edge_attr.shape[0]
    n0, n1, n2 = W0.shape[0], W1.shape[0], W2.shape[0]
    table = (W0[:, None, None, :] + W1[None, :, None, :]
             + W2[None, None, :, :]).reshape(n0 * n1 * n2, D_EMB)
    c0 = edge_attr[:, 0]
    c1 = edge_attr[:, 1]
    c2 = edge_attr[:, 2]
    return _bond_encode_sc(c0, c1, c2, table, E, n1, n2)


# speedup vs baseline: 1.0941x; 1.0102x over previous
import functools

import jax
import jax.numpy as jnp
from jax import lax
from jax.experimental import pallas as pl
from jax.experimental.pallas import tpu as pltpu
from jax.experimental.pallas import tpu_sc as plsc

D_EMB = 128
NUM_WORKERS = 32
CHUNK = 80
U = 5


def _bond_encode_sc(codes0, codes1, codes2, table, E, n1, n2):
    per_w = E // NUM_WORKERS
    n_bodies = per_w // (CHUNK * U)
    mesh = plsc.VectorSubcoreMesh(core_axis_name="c", subcore_axis_name="s")

    scratch = (
        [pltpu.VMEM((U, CHUNK), jnp.int32) for _ in range(3)]
        + [pltpu.VMEM((U, CHUNK), jnp.int32)]
        + [pltpu.VMEM((U, CHUNK, D_EMB), jnp.float32)]
        + [pltpu.SemaphoreType.DMA for _ in range(3 * U)]
    )

    @functools.partial(
        pl.kernel,
        mesh=mesh,
        out_type=jax.ShapeDtypeStruct((E, D_EMB), jnp.float32),
        scratch_types=scratch,
    )
    def k(c0_h, c1_h, c2_h, s_h, out_h, a0_v, a1_v, a2_v, idx_v, rows_v, *sems):
        sem_in = sems[0:U]
        sem_g = sems[U:2 * U]
        sem_cp = sems[2 * U:3 * U]
        wid = lax.axis_index("s") * 2 + lax.axis_index("c")
        base_w = wid * per_w

        def body(bi, carry):
            base0 = base_w + bi * (CHUNK * U)
            ins = []
            for u in range(U):
                base = base0 + u * CHUNK
                h0 = pltpu.async_copy(c0_h.at[pl.ds(base, CHUNK)], a0_v.at[u], sem_in[u])
                h1 = pltpu.async_copy(c1_h.at[pl.ds(base, CHUNK)], a1_v.at[u], sem_in[u])
                h2 = pltpu.async_copy(c2_h.at[pl.ds(base, CHUNK)], a2_v.at[u], sem_in[u])
                ins.append((h0, h1, h2))
            gs = []
            for u in range(U):
                for h in ins[u]:
                    h.wait()
                for i in range(CHUNK // 16):
                    s = pl.ds(i * 16, 16)
                    idx_v[u, s] = a0_v[u, s] * (n1 * n2) + a1_v[u, s] * n2 + a2_v[u, s]
                gs.append(pltpu.async_copy(s_h.at[idx_v.at[u]], rows_v.at[u], sem_g[u]))
            cps = []
            for u in range(U):
                gs[u].wait()
                base = base0 + u * CHUNK
                cps.append(pltpu.async_copy(rows_v.at[u], out_h.at[pl.ds(base, CHUNK)], sem_cp[u]))
            for u in range(U):
                cps[u].wait()
            return carry

        lax.fori_loop(0, n_bodies, body, 0)

    return k(codes0, codes1, codes2, table)


def kernel(edge_attr, W0, W1, W2):
    E = edge_attr.shape[0]
    n0, n1, n2 = W0.shape[0], W1.shape[0], W2.shape[0]
    table = (W0[:, None, None, :] + W1[None, :, None, :]
             + W2[None, None, :, :]).reshape(n0 * n1 * n2, D_EMB)
    c0 = edge_attr[:, 0]
    c1 = edge_attr[:, 1]
    c2 = edge_attr[:, 2]
    return _bond_encode_sc(c0, c1, c2, table, E, n1, n2)


# speedup vs baseline: 4.3511x; 3.9770x over previous
import functools

import jax
import jax.numpy as jnp
from jax import lax
from jax.experimental import pallas as pl
from jax.experimental.pallas import tpu as pltpu
from jax.experimental.pallas import tpu_sc as plsc

D_EMB = 128
NUM_WORKERS = 32
CHUNK = 80
U = 5


def _bond_encode_sc(codes0, codes1, codes2, table, E, n1, n2):
    per_w = E // NUM_WORKERS
    n_rows = table.shape[0] // NUM_WORKERS
    n_bodies = per_w // (CHUNK * U)
    mesh = plsc.VectorSubcoreMesh(core_axis_name="c", subcore_axis_name="s")

    scratch = (
        [pltpu.VMEM((U, CHUNK), jnp.int32) for _ in range(3)]
        + [pltpu.VMEM((U, CHUNK), jnp.int32)]
        + [pltpu.VMEM((U, CHUNK, D_EMB), jnp.float32)]
        + [pltpu.SemaphoreType.DMA for _ in range(3 * U)]
    )

    @functools.partial(
        pl.kernel,
        mesh=mesh,
        out_type=jax.ShapeDtypeStruct((E, D_EMB), jnp.float32),
        scratch_types=scratch,
    )
    def k(c0_h, c1_h, c2_h, s_h, out_h, a0_v, a1_v, a2_v, idx_v, rows_v, *sems):
        sem_in = sems[0:U]
        sem_g = sems[U:2 * U]
        sem_cp = sems[2 * U:3 * U]
        wid = lax.axis_index("s") * 2 + lax.axis_index("c")
        base_w = wid * per_w
        tab_off = wid * n_rows

        def body(bi, carry):
            base0 = base_w + bi * (CHUNK * U)
            ins = []
            for u in range(U):
                base = base0 + u * CHUNK
                h0 = pltpu.async_copy(c0_h.at[pl.ds(base, CHUNK)], a0_v.at[u], sem_in[u])
                h1 = pltpu.async_copy(c1_h.at[pl.ds(base, CHUNK)], a1_v.at[u], sem_in[u])
                h2 = pltpu.async_copy(c2_h.at[pl.ds(base, CHUNK)], a2_v.at[u], sem_in[u])
                ins.append((h0, h1, h2))
            gs = []
            for u in range(U):
                for h in ins[u]:
                    h.wait()
                for i in range(CHUNK // 16):
                    s = pl.ds(i * 16, 16)
                    idx_v[u, s] = (a0_v[u, s] * (n1 * n2) + a1_v[u, s] * n2
                                   + a2_v[u, s] + tab_off)
                gs.append(pltpu.async_copy(s_h.at[idx_v.at[u]], rows_v.at[u], sem_g[u]))
            cps = []
            for u in range(U):
                gs[u].wait()
                base = base0 + u * CHUNK
                cps.append(pltpu.async_copy(rows_v.at[u], out_h.at[pl.ds(base, CHUNK)], sem_cp[u]))
            for u in range(U):
                cps[u].wait()
            return carry

        lax.fori_loop(0, n_bodies, body, 0)

    return k(codes0, codes1, codes2, table)


def kernel(edge_attr, W0, W1, W2):
    E = edge_attr.shape[0]
    n0, n1, n2 = W0.shape[0], W1.shape[0], W2.shape[0]
    table = (W0[:, None, None, :] + W1[None, :, None, :]
             + W2[None, None, :, :]).reshape(n0 * n1 * n2, D_EMB)
    table = jnp.tile(table, (NUM_WORKERS, 1))
    c0 = edge_attr[:, 0]
    c1 = edge_attr[:, 1]
    c2 = edge_attr[:, 2]
    return _bond_encode_sc(c0, c1, c2, table, E, n1, n2)


# speedup vs baseline: 8.1474x; 1.8725x over previous
import functools

import jax
import jax.numpy as jnp
from jax import lax
from jax.experimental import pallas as pl
from jax.experimental.pallas import tpu as pltpu
from jax.experimental.pallas import tpu_sc as plsc

D_EMB = 128
NUM_WORKERS = 32
CHUNK = 80
U = 5


def _bond_encode_sc(codes0, codes1, codes2, table, E, n1, n2):
    per_w = E // NUM_WORKERS
    n_rows = table.shape[0] // (NUM_WORKERS * U)
    n_bodies = per_w // (CHUNK * U)
    mesh = plsc.VectorSubcoreMesh(core_axis_name="c", subcore_axis_name="s")

    scratch = (
        [pltpu.VMEM((U, CHUNK), jnp.int32) for _ in range(3)]
        + [pltpu.VMEM((U, CHUNK), jnp.int32)]
        + [pltpu.VMEM((U, CHUNK, D_EMB), jnp.float32)]
        + [pltpu.SemaphoreType.DMA for _ in range(3 * U)]
    )

    @functools.partial(
        pl.kernel,
        mesh=mesh,
        out_type=jax.ShapeDtypeStruct((E, D_EMB), jnp.float32),
        scratch_types=scratch,
    )
    def k(c0_h, c1_h, c2_h, s_h, out_h, a0_v, a1_v, a2_v, idx_v, rows_v, *sems):
        sem_in = sems[0:U]
        sem_g = sems[U:2 * U]
        sem_cp = sems[2 * U:3 * U]
        wid = lax.axis_index("s") * 2 + lax.axis_index("c")
        base_w = wid * per_w
        tab_off = wid * (U * n_rows)

        def body(bi, carry):
            base0 = base_w + bi * (CHUNK * U)
            ins = []
            for u in range(U):
                base = base0 + u * CHUNK
                h0 = pltpu.async_copy(c0_h.at[pl.ds(base, CHUNK)], a0_v.at[u], sem_in[u])
                h1 = pltpu.async_copy(c1_h.at[pl.ds(base, CHUNK)], a1_v.at[u], sem_in[u])
                h2 = pltpu.async_copy(c2_h.at[pl.ds(base, CHUNK)], a2_v.at[u], sem_in[u])
                ins.append((h0, h1, h2))
            gs = []
            for u in range(U):
                for h in ins[u]:
                    h.wait()
                for i in range(CHUNK // 16):
                    s = pl.ds(i * 16, 16)
                    idx_v[u, s] = (a0_v[u, s] * (n1 * n2) + a1_v[u, s] * n2
                                   + a2_v[u, s] + (tab_off + u * n_rows))
                gs.append(pltpu.async_copy(s_h.at[idx_v.at[u]], rows_v.at[u], sem_g[u]))
            cps = []
            for u in range(U):
                gs[u].wait()
                base = base0 + u * CHUNK
                cps.append(pltpu.async_copy(rows_v.at[u], out_h.at[pl.ds(base, CHUNK)], sem_cp[u]))
            for u in range(U):
                cps[u].wait()
            return carry

        lax.fori_loop(0, n_bodies, body, 0)

    return k(codes0, codes1, codes2, table)


def kernel(edge_attr, W0, W1, W2):
    E = edge_attr.shape[0]
    n0, n1, n2 = W0.shape[0], W1.shape[0], W2.shape[0]
    table = (W0[:, None, None, :] + W1[None, :, None, :]
             + W2[None, None, :, :]).reshape(n0 * n1 * n2, D_EMB)
    table = jnp.tile(table, (NUM_WORKERS * U, 1))
    c0 = edge_attr[:, 0]
    c1 = edge_attr[:, 1]
    c2 = edge_attr[:, 2]
    return _bond_encode_sc(c0, c1, c2, table, E, n1, n2)
